# baseline (speedup 1.0000x reference)
import functools

import jax
import jax.numpy as jnp
from jax import lax
from jax.experimental import pallas as pl
from jax.experimental.pallas import tpu as pltpu
from jax.experimental.pallas import tpu_sc as plsc

NUM_NODES = 1000000
NUM_HIGH = 100000
NUM_LOW = NUM_NODES - NUM_HIGH
D_HIGH = 128
D_LOW = 32
D_COMMON = 64
B = 16384

NC = 2
NS = 16
NW = NC * NS
B_PER_W = B // NW
IDX_CHUNK = 128
N_CHUNKS = B_PER_W // IDX_CHUNK


def _sc_gather(high_idx, low_idx, emb_high, emb_low):
    mesh = plsc.VectorSubcoreMesh(
        core_axis_name="c", subcore_axis_name="s", num_cores=NC, num_subcores=NS
    )

    @functools.partial(
        pl.kernel,
        out_type=(
            jax.ShapeDtypeStruct((B, D_HIGH), jnp.float32),
            jax.ShapeDtypeStruct((B, D_LOW), jnp.float32),
        ),
        mesh=mesh,
        compiler_params=pltpu.CompilerParams(use_tc_tiling_on_sc=False),
        scratch_types=[
            pltpu.VMEM((N_CHUNKS, IDX_CHUNK), jnp.int32),
            pltpu.VMEM((N_CHUNKS, IDX_CHUNK), jnp.int32),
            pltpu.VMEM((B_PER_W, D_HIGH), jnp.float32),
            pltpu.VMEM((B_PER_W, D_LOW), jnp.float32),
            pltpu.SemaphoreType.DMA,
        ],
    )
    def k(hidx_hbm, lidx_hbm, eh_hbm, el_hbm, gh_hbm, gl_hbm,
          hidx_v, lidx_v, rows_h, rows_l, sem):
        wid = lax.axis_index("s") * NC + lax.axis_index("c")
        base = wid * B_PER_W
        pltpu.sync_copy(hidx_hbm.at[wid], hidx_v)
        pltpu.sync_copy(lidx_hbm.at[wid], lidx_v)
        copies = []
        for j in range(N_CHUNKS):
            copies.append(pltpu.async_copy(
                eh_hbm.at[hidx_v.at[j]],
                rows_h.at[pl.ds(j * IDX_CHUNK, IDX_CHUNK)], sem))
            copies.append(pltpu.async_copy(
                el_hbm.at[lidx_v.at[j]],
                rows_l.at[pl.ds(j * IDX_CHUNK, IDX_CHUNK)], sem))
        for c in copies:
            c.wait()
        pltpu.sync_copy(rows_h, gh_hbm.at[pl.ds(base, B_PER_W)])
        pltpu.sync_copy(rows_l, gl_hbm.at[pl.ds(base, B_PER_W)])

    return k(high_idx.reshape(NW, N_CHUNKS, IDX_CHUNK),
             low_idx.reshape(NW, N_CHUNKS, IDX_CHUNK),
             emb_high, emb_low)


BLK = 2048


def _tc_body(ids_ref, gh_ref, gl_ref, wh_ref, bh_ref, wl_ref, bl_ref, out_ref):
    h = lax.dot_general(gh_ref[...], wh_ref[...],
                        (((1,), (1,)), ((), ())),
                        preferred_element_type=jnp.float32) + bh_ref[...]
    l = lax.dot_general(gl_ref[...], wl_ref[...],
                        (((1,), (1,)), ((), ())),
                        preferred_element_type=jnp.float32) + bl_ref[...]
    out_ref[...] = jnp.where(ids_ref[...] < NUM_HIGH, h, l)


def _tc_project(node_ids, gh, gl, W_high, b_high, W_low, b_low):
    grid = (B // BLK,)
    return pl.pallas_call(
        _tc_body,
        grid=grid,
        in_specs=[
            pl.BlockSpec((BLK, 1), lambda i: (i, 0)),
            pl.BlockSpec((BLK, D_HIGH), lambda i: (i, 0)),
            pl.BlockSpec((BLK, D_LOW), lambda i: (i, 0)),
            pl.BlockSpec((D_COMMON, D_HIGH), lambda i: (0, 0)),
            pl.BlockSpec((1, D_COMMON), lambda i: (0, 0)),
            pl.BlockSpec((D_COMMON, D_LOW), lambda i: (0, 0)),
            pl.BlockSpec((1, D_COMMON), lambda i: (0, 0)),
        ],
        out_specs=pl.BlockSpec((BLK, D_COMMON), lambda i: (i, 0)),
        out_shape=jax.ShapeDtypeStruct((B, D_COMMON), jnp.float32),
    )(node_ids.reshape(B, 1), gh, gl, W_high,
      b_high.reshape(1, D_COMMON), W_low, b_low.reshape(1, D_COMMON))


def kernel(node_ids, emb_high, emb_low, W_high, b_high, W_low, b_low):
    is_high = node_ids < NUM_HIGH
    high_idx = jnp.where(is_high, node_ids, 0)
    low_idx = jnp.where(is_high, 0,
                        jnp.clip(node_ids - NUM_HIGH, 0, NUM_LOW - 1))
    gh, gl = _sc_gather(high_idx, low_idx, emb_high, emb_low)
    return _tc_project(node_ids, gh, gl, W_high, b_high, W_low, b_low)
